# Initial kernel scaffold; baseline (speedup 1.0000x reference)
import jax
import jax.numpy as jnp
from jax.experimental import pallas as pl


def kernel(x, A_r, S_r, b_r, A1, S1, b1, W2, b2):
    raise NotImplementedError("write your pallas kernel here")


# speedup vs baseline: 2.4419x; 2.4419x over previous
import functools

import jax
import jax.numpy as jnp
from jax.experimental import pallas as pl
from jax.experimental.pallas import tpu as pltpu

NN = 4
DIM = 768
E = 8
TOPK = 2
EXPERT_DIM = 3072
SEQ = 2048
DC = DIM // NN
FC = EXPERT_DIM // NN

TN = 1024
NT = SEQ // TN


def _router_body(x_ref, wrt_ref, br_ref, gates_ref):
    logits = jax.lax.dot_general(
        x_ref[...], wrt_ref[...], (((1,), (0,)), ((), ())),
        preferred_element_type=jnp.float32) + br_ref[...]
    tn = logits.shape[0]
    iota = jax.lax.broadcasted_iota(jnp.int32, (tn, E), 1)
    m1 = jnp.max(logits, axis=1, keepdims=True)
    am1 = jnp.min(jnp.where(logits == m1, iota, E), axis=1, keepdims=True)
    masked = jnp.where(iota == am1, -jnp.inf, logits)
    m2 = jnp.max(masked, axis=1, keepdims=True)
    am2 = jnp.min(jnp.where(masked == m2, iota, E), axis=1, keepdims=True)
    w1 = 1.0 / (1.0 + jnp.exp(m2 - m1))
    w2 = 1.0 - w1
    gates_ref[...] = (jnp.where(iota == am1, w1, 0.0)
                      + jnp.where(iota == am2, w2, 0.0))


def _expert_body(a1_ref, x_ref, s1cat_ref, b1_ref, w2_ref, b2_ref, gates_ref,
                 out_ref):
    e = pl.program_id(1)
    x = x_ref[...]
    s1cat = s1cat_ref[0]
    h_blocks = []
    for a in range(NN):
        xc_parts = []
        for i in range(NN):
            acc = a1_ref[0, i, a, 0] * x[:, 0:DC]
            for b in range(1, NN):
                acc = acc + a1_ref[0, i, a, b] * x[:, b * DC:(b + 1) * DC]
            xc_parts.append(acc)
        xc = jnp.concatenate(xc_parts, axis=1)
        ha = jax.lax.dot_general(xc, s1cat, (((1,), (0,)), ((), ())),
                                 preferred_element_type=jnp.float32)
        h_blocks.append(ha)
    h = jnp.concatenate(h_blocks, axis=1) + b1_ref[0, 0]
    h = 0.5 * h * (1.0 + jax.lax.erf(h * (2.0 ** -0.5)))
    y = jax.lax.dot_general(h, w2_ref[0], (((1,), (1,)), ((), ())),
                            preferred_element_type=jnp.float32) + b2_ref[0, 0]
    iota = jax.lax.broadcasted_iota(jnp.int32, (x.shape[0], E), 1)
    g = jnp.sum(jnp.where(iota == e, gates_ref[...], 0.0), axis=1,
                keepdims=True)
    gy = g * y

    @pl.when(e == 0)
    def _():
        out_ref[...] = gy

    @pl.when(e > 0)
    def _():
        out_ref[...] = out_ref[...] + gy


def kernel(x, A_r, S_r, b_r, A1, S1, b1, W2, b2):
    x2d = x.reshape(SEQ, DIM)
    wr = jnp.sum(
        jnp.einsum('iab,icd->iacbd', A_r, S_r).reshape(NN, E, DIM), axis=0)
    wrt = wr.T
    gates = pl.pallas_call(
        _router_body,
        grid=(NT,),
        in_specs=[
            pl.BlockSpec((TN, DIM), lambda t: (t, 0)),
            pl.BlockSpec((DIM, E), lambda t: (0, 0)),
            pl.BlockSpec((1, E), lambda t: (0, 0)),
        ],
        out_specs=pl.BlockSpec((TN, E), lambda t: (t, 0)),
        out_shape=jax.ShapeDtypeStruct((SEQ, E), jnp.float32),
    )(x2d, wrt, b_r.reshape(1, E))

    s1cat = jnp.transpose(S1, (0, 1, 3, 2)).reshape(E, DIM, FC)

    out2d = pl.pallas_call(
        _expert_body,
        grid=(NT, E),
        in_specs=[
            pl.BlockSpec((1, NN, NN, NN), lambda t, e: (e, 0, 0, 0),
                         memory_space=pltpu.SMEM),
            pl.BlockSpec((TN, DIM), lambda t, e: (t, 0)),
            pl.BlockSpec((1, DIM, FC), lambda t, e: (e, 0, 0)),
            pl.BlockSpec((1, 1, EXPERT_DIM), lambda t, e: (e, 0, 0)),
            pl.BlockSpec((1, DIM, EXPERT_DIM), lambda t, e: (e, 0, 0)),
            pl.BlockSpec((1, 1, DIM), lambda t, e: (e, 0, 0)),
            pl.BlockSpec((TN, E), lambda t, e: (t, 0)),
        ],
        out_specs=pl.BlockSpec((TN, DIM), lambda t, e: (t, 0)),
        out_shape=jax.ShapeDtypeStruct((SEQ, DIM), jnp.float32),
    )(A1, x2d, s1cat, b1.reshape(E, 1, EXPERT_DIM), W2,
      b2.reshape(E, 1, DIM), gates)
    return out2d.reshape(x.shape)
